# baseline (speedup 1.0000x reference)
import functools

import jax
import jax.numpy as jnp
from jax.experimental import pallas as pl
from jax.experimental.pallas import tpu as pltpu

_VMEM_LIMIT_BYTES = 100 * 1024 * 1024


def _gcn_kernel(adj_ref, w1_ref, wm_ref, w2_ref, out_ref,
                adjb_ref, w1b_ref, wmb_ref, w2b_ref, pre1_ref,
                *, tm):
    p = pl.program_id(0)
    j = pl.program_id(1)
    rows = pl.ds(j * tm, tm)

    @pl.when((p == 0) & (j == 0))
    def _():
        w1b_ref[...] = w1_ref[...].astype(jnp.bfloat16)
        wmb_ref[...] = wm_ref[...].astype(jnp.bfloat16)
        w2b_ref[...] = w2_ref[...].astype(jnp.bfloat16)

    @pl.when(p == 0)
    def _():
        a = adj_ref[...].astype(jnp.bfloat16)
        adjb_ref[rows, :] = a
        h = jnp.dot(a, w1b_ref[...], preferred_element_type=jnp.float32)
        h = jnp.maximum(h, 0.0).astype(jnp.bfloat16)
        pre1_ref[rows, :] = jnp.dot(
            h, wmb_ref[...], preferred_element_type=jnp.float32
        ).astype(jnp.bfloat16)

    @pl.when(p == 1)
    def _():
        h = jnp.dot(adjb_ref[rows, :], pre1_ref[...],
                    preferred_element_type=jnp.float32)
        h = jnp.maximum(h, 0.0).astype(jnp.bfloat16)
        pre2 = jnp.dot(
            h, w2b_ref[...], preferred_element_type=jnp.float32
        ).astype(jnp.bfloat16)
        partial = jnp.dot(adjb_ref[:, rows], pre2,
                          preferred_element_type=jnp.float32)

        @pl.when(j == 0)
        def _():
            out_ref[...] = partial

        @pl.when(j > 0)
        def _():
            out_ref[...] += partial


def kernel(W1, W2, Wmid0, x, adj):
    del x
    n, k = adj.shape
    h = W1.shape[1]
    c = W2.shape[1]
    tm = min(256, n)
    s = n // tm
    assert n % tm == 0, adj.shape

    adj_idx = lambda p, j: (jnp.where(p == 0, j, s - 1), 0)
    const = lambda p, j: (0, 0)

    return pl.pallas_call(
        functools.partial(_gcn_kernel, tm=tm),
        out_shape=jax.ShapeDtypeStruct((n, c), jnp.float32),
        grid=(2, s),
        in_specs=[
            pl.BlockSpec((tm, k), adj_idx),
            pl.BlockSpec((k, h), const),
            pl.BlockSpec((h, h), const),
            pl.BlockSpec((h, c), const),
        ],
        out_specs=pl.BlockSpec((n, c), const),
        scratch_shapes=[
            pltpu.VMEM((n, k), jnp.bfloat16),
            pltpu.VMEM((k, h), jnp.bfloat16),
            pltpu.VMEM((h, h), jnp.bfloat16),
            pltpu.VMEM((h, c), jnp.bfloat16),
            pltpu.VMEM((n, h), jnp.bfloat16),
        ],
        compiler_params=pltpu.CompilerParams(
            dimension_semantics=("arbitrary", "arbitrary"),
            vmem_limit_bytes=_VMEM_LIMIT_BYTES,
        ),
    )(adj, W1, Wmid0, W2)


# speedup vs baseline: 1.1736x; 1.1736x over previous
import functools

import jax
import jax.numpy as jnp
from jax.experimental import pallas as pl
from jax.experimental.pallas import tpu as pltpu

_VMEM_LIMIT_BYTES = 100 * 1024 * 1024


def _gcn_kernel(adj_ref, w1_ref, wm_ref, w2_ref, out_ref,
                adjb_ref, w1b_ref, wmb_ref, w2b_ref, pre1_ref,
                *, tm):
    p = pl.program_id(0)
    j = pl.program_id(1)
    rows = pl.ds(j * tm, tm)

    @pl.when((p == 0) & (j == 0))
    def _():
        w1b_ref[...] = w1_ref[...].astype(jnp.bfloat16)
        wmb_ref[...] = wm_ref[...].astype(jnp.bfloat16)
        w2b_ref[...] = w2_ref[...].astype(jnp.bfloat16)

    @pl.when(p == 0)
    def _():
        a = adj_ref[...].astype(jnp.bfloat16)
        adjb_ref[rows, :] = a
        h = jnp.dot(a, w1b_ref[...], preferred_element_type=jnp.float32)
        h = jnp.maximum(h, 0.0).astype(jnp.bfloat16)
        pre1_ref[rows, :] = jnp.dot(
            h, wmb_ref[...], preferred_element_type=jnp.float32
        ).astype(jnp.bfloat16)

    @pl.when(p == 1)
    def _():
        h = jnp.dot(adjb_ref[rows, :], pre1_ref[...],
                    preferred_element_type=jnp.float32)
        h = jnp.maximum(h, 0.0).astype(jnp.bfloat16)
        pre2 = jnp.dot(
            h, w2b_ref[...], preferred_element_type=jnp.float32
        ).astype(jnp.bfloat16)
        partial = jnp.dot(adjb_ref[:, rows], pre2,
                          preferred_element_type=jnp.float32)

        @pl.when(j == 0)
        def _():
            out_ref[...] = partial

        @pl.when(j > 0)
        def _():
            out_ref[...] += partial


def kernel(W1, W2, Wmid0, x, adj):
    del x
    n, k = adj.shape
    h = W1.shape[1]
    c = W2.shape[1]
    tm = min(1024, n)
    s = n // tm
    assert n % tm == 0, adj.shape

    adj_idx = lambda p, j: (jnp.where(p == 0, j, s - 1), 0)
    const = lambda p, j: (0, 0)

    return pl.pallas_call(
        functools.partial(_gcn_kernel, tm=tm),
        out_shape=jax.ShapeDtypeStruct((n, c), jnp.float32),
        grid=(2, s),
        in_specs=[
            pl.BlockSpec((tm, k), adj_idx),
            pl.BlockSpec((k, h), const),
            pl.BlockSpec((h, h), const),
            pl.BlockSpec((h, c), const),
        ],
        out_specs=pl.BlockSpec((n, c), const),
        scratch_shapes=[
            pltpu.VMEM((n, k), jnp.bfloat16),
            pltpu.VMEM((k, h), jnp.bfloat16),
            pltpu.VMEM((h, h), jnp.bfloat16),
            pltpu.VMEM((h, c), jnp.bfloat16),
            pltpu.VMEM((n, h), jnp.bfloat16),
        ],
        compiler_params=pltpu.CompilerParams(
            dimension_semantics=("arbitrary", "arbitrary"),
            vmem_limit_bytes=_VMEM_LIMIT_BYTES,
        ),
    )(adj, W1, Wmid0, W2)


# speedup vs baseline: 1.1992x; 1.0218x over previous
import functools

import jax
import jax.numpy as jnp
from jax.experimental import pallas as pl
from jax.experimental.pallas import tpu as pltpu

_VMEM_LIMIT_BYTES = 100 * 1024 * 1024


def _gcn_kernel(adj_ref, w1_ref, wm_ref, w2_ref, out_ref,
                adjb_ref, w1b_ref, wmb_ref, w2b_ref, pre1_ref, pre2_ref,
                *, tm):
    p = pl.program_id(0)
    j = pl.program_id(1)
    rows = pl.ds(j * tm, tm)

    @pl.when((p == 0) & (j == 0))
    def _():
        w1b_ref[...] = w1_ref[...].astype(jnp.bfloat16)
        wmb_ref[...] = wm_ref[...].astype(jnp.bfloat16)
        w2b_ref[...] = w2_ref[...].astype(jnp.bfloat16)

    @pl.when(p == 0)
    def _():
        a = adj_ref[...].astype(jnp.bfloat16)
        adjb_ref[rows, :] = a
        h = jnp.dot(a, w1b_ref[...], preferred_element_type=jnp.float32)
        h = jnp.maximum(h, 0.0).astype(jnp.bfloat16)
        pre1_ref[rows, :] = jnp.dot(
            h, wmb_ref[...], preferred_element_type=jnp.float32
        ).astype(jnp.bfloat16)

    @pl.when(p == 1)
    def _():
        h = jnp.dot(adjb_ref[rows, :], pre1_ref[...],
                    preferred_element_type=jnp.float32)
        h = jnp.maximum(h, 0.0).astype(jnp.bfloat16)
        pre2_ref[rows, :] = jnp.dot(
            h, w2b_ref[...], preferred_element_type=jnp.float32
        ).astype(jnp.bfloat16)

    @pl.when(p == 2)
    def _():
        out_ref[...] = jnp.dot(adjb_ref[rows, :], pre2_ref[...],
                               preferred_element_type=jnp.float32)


def kernel(W1, W2, Wmid0, x, adj):
    del x
    n, k = adj.shape
    h = W1.shape[1]
    c = W2.shape[1]
    tm = min(512, n)
    s = n // tm
    assert n % tm == 0, adj.shape

    adj_idx = lambda p, j: (jnp.where(p == 0, j, s - 1), 0)
    const = lambda p, j: (0, 0)

    return pl.pallas_call(
        functools.partial(_gcn_kernel, tm=tm),
        out_shape=jax.ShapeDtypeStruct((n, c), jnp.float32),
        grid=(3, s),
        in_specs=[
            pl.BlockSpec((tm, k), adj_idx),
            pl.BlockSpec((k, h), const),
            pl.BlockSpec((h, h), const),
            pl.BlockSpec((h, c), const),
        ],
        out_specs=pl.BlockSpec((tm, c), lambda p, j: (j, 0)),
        scratch_shapes=[
            pltpu.VMEM((n, k), jnp.bfloat16),
            pltpu.VMEM((k, h), jnp.bfloat16),
            pltpu.VMEM((h, h), jnp.bfloat16),
            pltpu.VMEM((h, c), jnp.bfloat16),
            pltpu.VMEM((n, h), jnp.bfloat16),
            pltpu.VMEM((n, c), jnp.bfloat16),
        ],
        compiler_params=pltpu.CompilerParams(
            dimension_semantics=("arbitrary", "arbitrary"),
            vmem_limit_bytes=_VMEM_LIMIT_BYTES,
        ),
    )(adj, W1, Wmid0, W2)


# speedup vs baseline: 1.2031x; 1.0033x over previous
import functools

import jax
import jax.numpy as jnp
from jax.experimental import pallas as pl
from jax.experimental.pallas import tpu as pltpu

_VMEM_LIMIT_BYTES = 100 * 1024 * 1024


def _gcn_kernel(adj_ref, w1_ref, wm_ref, w2_ref, out_ref,
                adjb_ref, w1b_ref, wmb_ref, w2b_ref, pre1_ref, pre2_ref,
                *, tm):
    p = pl.program_id(0)
    j = pl.program_id(1)
    rows = pl.ds(j * tm, tm)

    @pl.when((p == 0) & (j == 0))
    def _():
        w1b_ref[...] = w1_ref[...].astype(jnp.bfloat16)
        wmb_ref[...] = wm_ref[...].astype(jnp.bfloat16)
        w2b_ref[...] = w2_ref[...].astype(jnp.bfloat16)

    @pl.when(p == 0)
    def _():
        a = adj_ref[...].astype(jnp.bfloat16)
        adjb_ref[rows, :] = a
        h = jnp.dot(a, w1b_ref[...], preferred_element_type=jnp.float32)
        h = jnp.maximum(h, 0.0).astype(jnp.bfloat16)
        pre1_ref[rows, :] = jnp.dot(
            h, wmb_ref[...], preferred_element_type=jnp.float32
        ).astype(jnp.bfloat16)

    @pl.when(p == 1)
    def _():
        h = jnp.dot(adjb_ref[rows, :], pre1_ref[...],
                    preferred_element_type=jnp.float32)
        h = jnp.maximum(h, 0.0).astype(jnp.bfloat16)
        pre2_ref[rows, :] = jnp.dot(
            h, w2b_ref[...], preferred_element_type=jnp.float32
        ).astype(jnp.bfloat16)

    @pl.when(p == 2)
    def _():
        out_ref[...] = jnp.dot(adjb_ref[rows, :], pre2_ref[...],
                               preferred_element_type=jnp.float32)


def kernel(W1, W2, Wmid0, x, adj):
    del x
    n, k = adj.shape
    h = W1.shape[1]
    c = W2.shape[1]
    tm = min(512, n)
    s = n // tm
    assert n % tm == 0, adj.shape

    adj_idx = lambda p, j: (jnp.where(p == 0, j, s - 1), 0)
    const = lambda p, j: (0, 0)

    return pl.pallas_call(
        functools.partial(_gcn_kernel, tm=tm),
        out_shape=jax.ShapeDtypeStruct((n, c), jnp.float32),
        grid=(3, s),
        in_specs=[
            pl.BlockSpec((tm, k), adj_idx),
            pl.BlockSpec((k, h), const),
            pl.BlockSpec((h, h), const),
            pl.BlockSpec((h, c), const),
        ],
        out_specs=pl.BlockSpec((tm, c), lambda p, j: (jnp.where(p == 2, j, 0), 0)),
        scratch_shapes=[
            pltpu.VMEM((n, k), jnp.bfloat16),
            pltpu.VMEM((k, h), jnp.bfloat16),
            pltpu.VMEM((h, h), jnp.bfloat16),
            pltpu.VMEM((h, c), jnp.bfloat16),
            pltpu.VMEM((n, h), jnp.bfloat16),
            pltpu.VMEM((n, c), jnp.bfloat16),
        ],
        compiler_params=pltpu.CompilerParams(
            dimension_semantics=("arbitrary", "arbitrary"),
            vmem_limit_bytes=_VMEM_LIMIT_BYTES,
        ),
    )(adj, W1, Wmid0, W2)


# speedup vs baseline: 1.2608x; 1.0479x over previous
import functools

import jax
import jax.numpy as jnp
from jax.experimental import pallas as pl
from jax.experimental.pallas import tpu as pltpu

_VMEM_LIMIT_BYTES = 100 * 1024 * 1024


def _gcn_kernel(adj_ref, w1_ref, wm_ref, w2_ref, out_ref,
                adjb_ref, w1b_ref, wmb_ref, w2b_ref, pre1_ref, pre2_ref,
                *, tm):
    p = pl.program_id(0)
    j = pl.program_id(1)
    rows = pl.ds(j * tm, tm)

    @pl.when((p == 0) & (j == 0))
    def _():
        w1b_ref[...] = w1_ref[...].astype(jnp.bfloat16)
        wmb_ref[...] = wm_ref[...].astype(jnp.bfloat16)
        w2b_ref[...] = w2_ref[...].astype(jnp.bfloat16)

    @pl.when(p == 0)
    def _():
        a = adj_ref[...].astype(jnp.bfloat16)
        adjb_ref[rows, :] = a
        h = jnp.dot(a, w1b_ref[...], preferred_element_type=jnp.float32)
        h = jnp.maximum(h, 0.0).astype(jnp.bfloat16)
        pre1_ref[rows, :] = jnp.dot(
            h, wmb_ref[...], preferred_element_type=jnp.float32
        ).astype(jnp.bfloat16)

    @pl.when(p == 1)
    def _():
        h = jnp.dot(adjb_ref[rows, :], pre1_ref[...],
                    preferred_element_type=jnp.float32)
        h = jnp.maximum(h, 0.0).astype(jnp.bfloat16)
        pre2_ref[rows, :] = jnp.dot(
            h, w2b_ref[...], preferred_element_type=jnp.float32
        ).astype(jnp.bfloat16)

    @pl.when(p == 2)
    def _():
        out_ref[...] = jnp.dot(adjb_ref[rows, :], pre2_ref[...],
                               preferred_element_type=jnp.float32)


def kernel(W1, W2, Wmid0, x, adj):
    del x
    n, k = adj.shape
    h = W1.shape[1]
    c = W2.shape[1]
    tm = min(1024, n)
    s = n // tm
    assert n % tm == 0, adj.shape

    adj_idx = lambda p, j: (jnp.where(p == 0, j, s - 1), 0)
    const = lambda p, j: (0, 0)

    return pl.pallas_call(
        functools.partial(_gcn_kernel, tm=tm),
        out_shape=jax.ShapeDtypeStruct((n, c), jnp.float32),
        grid=(3, s),
        in_specs=[
            pl.BlockSpec((tm, k), adj_idx),
            pl.BlockSpec((k, h), const),
            pl.BlockSpec((h, h), const),
            pl.BlockSpec((h, c), const),
        ],
        out_specs=pl.BlockSpec((tm, c), lambda p, j: (jnp.where(p == 2, j, 0), 0)),
        scratch_shapes=[
            pltpu.VMEM((n, k), jnp.bfloat16),
            pltpu.VMEM((k, h), jnp.bfloat16),
            pltpu.VMEM((h, h), jnp.bfloat16),
            pltpu.VMEM((h, c), jnp.bfloat16),
            pltpu.VMEM((n, h), jnp.bfloat16),
            pltpu.VMEM((n, c), jnp.bfloat16),
        ],
        compiler_params=pltpu.CompilerParams(
            dimension_semantics=("arbitrary", "arbitrary"),
            vmem_limit_bytes=_VMEM_LIMIT_BYTES,
        ),
    )(adj, W1, Wmid0, W2)
